# baseline (speedup 1.0000x reference)
import functools

import jax
import jax.numpy as jnp
from jax import lax
from jax.experimental import pallas as pl
from jax.experimental.pallas import tpu as pltpu
from jax.experimental.pallas import tpu_sc as plsc

_F32 = jnp.float32
_VMEM_LIMIT = 100 * 1024 * 1024


def _row_tile(h):
    return {24: 24, 48: 16, 96: 16, 192: 8, 384: 8, 193: 1, 97: 1, 49: 1, 25: 1}[h]


def _conv_mm(xp, w_taps, bias, alpha, mask, kh, kw, hout, wout):
    cin = xp.shape[-1]
    cout = w_taps.shape[-1]
    th = _row_tile(hout)
    grid = (hout // th,)
    has_b = bias is not None
    has_a = alpha is not None
    has_m = mask is not None
    one_by_one = kh == 1 and kw == 1

    def body(*refs):
        x_ref, w_ref = refs[0], refs[1]
        k = 2
        b_ref = a_ref = m_ref = None
        if has_b:
            b_ref = refs[k]; k += 1
        if has_a:
            a_ref = refs[k]; k += 1
        if has_m:
            m_ref = refs[k]; k += 1
        o_ref = refs[k]
        i = pl.program_id(0)
        acc = jnp.zeros((th * wout, cout), _F32)
        t = 0
        for dy in range(kh):
            for dx in range(kw):
                if one_by_one:
                    xs = x_ref[...]
                else:
                    xs = x_ref[pl.ds(i * th + dy, th), dx:dx + wout, :]
                acc = acc + jnp.dot(xs.reshape(th * wout, cin), w_ref[t],
                                    preferred_element_type=_F32)
                t += 1
        if has_b:
            acc = acc + b_ref[...]
        if has_a:
            acc = jnp.where(acc >= 0, acc, acc * a_ref[...])
        y = acc.reshape(th, wout, cout)
        if has_m:
            y = y * m_ref[...][:, :, None]
        o_ref[...] = y

    if one_by_one:
        x_spec = pl.BlockSpec((th, wout, cin), lambda i: (i, 0, 0))
    else:
        x_spec = pl.BlockSpec(xp.shape, lambda i: (0, 0, 0))
    in_specs = [x_spec, pl.BlockSpec(w_taps.shape, lambda i: (0, 0, 0))]
    ops = [xp, w_taps]
    if has_b:
        ops.append(bias.reshape(1, cout))
        in_specs.append(pl.BlockSpec((1, cout), lambda i: (0, 0)))
    if has_a:
        ops.append(alpha.reshape(1, cout))
        in_specs.append(pl.BlockSpec((1, cout), lambda i: (0, 0)))
    if has_m:
        ops.append(mask)
        in_specs.append(pl.BlockSpec((th, wout), lambda i: (i, 0)))
    return pl.pallas_call(
        body,
        grid=grid,
        in_specs=in_specs,
        out_specs=pl.BlockSpec((th, wout, cout), lambda i: (i, 0, 0)),
        out_shape=jax.ShapeDtypeStruct((hout, wout, cout), _F32),
        compiler_params=pltpu.CompilerParams(vmem_limit_bytes=_VMEM_LIMIT),
    )(*ops)


def _conv3(x, w, b=None, a=None, mask=None):
    h, wd, cin = x.shape
    cout = w.shape[0]
    xp = jnp.pad(x, ((1, 1), (1, 1), (0, 0)))
    wt = jnp.transpose(w, (2, 3, 1, 0)).reshape(9, cin, cout)
    nb = 4 if h >= 384 else 1
    if nb == 1:
        return _conv_mm(xp, wt, b, a, mask, 3, 3, h, wd)
    hb = h // nb
    outs = []
    for bi in range(nb):
        xb = xp[bi * hb:bi * hb + hb + 2]
        mb = mask[bi * hb:(bi + 1) * hb] if mask is not None else None
        outs.append(_conv_mm(xb, wt, b, a, mb, 3, 3, hb, wd))
    return jnp.concatenate(outs, 0)


def _conv3_s2(x, w, b, a):
    h, wd, cin = x.shape
    cout = w.shape[0]
    xp = jnp.pad(x, ((1, 1), (1, 1), (0, 0)))
    hp, wp = (h + 2) // 2, (wd + 2) // 2
    s2d = xp.reshape(hp, 2, wp, 2, cin).transpose(0, 2, 1, 3, 4).reshape(hp, wp, 4 * cin)
    wpad = jnp.pad(w, ((0, 0), (0, 0), (0, 1), (0, 1)))
    wt = wpad.reshape(cout, cin, 2, 2, 2, 2).transpose(2, 4, 3, 5, 1, 0)
    wt = wt.reshape(4, 4 * cin, cout)
    return _conv_mm(s2d, wt, b, a, None, 2, 2, h // 2, wd // 2)


def _warp_prep(flow_t, h, w):
    hw = h * w

    def body(f_ref, idx_ref, wgt_ref):
        gx = lax.broadcasted_iota(jnp.int32, (h, w), 1).astype(_F32)
        gy = lax.broadcasted_iota(jnp.int32, (h, w), 0).astype(_F32)
        for f in range(2):
            fx = f_ref[2 * f]
            fy = f_ref[2 * f + 1]
            xc = gx + fx
            yc = gy + fy
            x0 = jnp.floor(xc)
            y0 = jnp.floor(yc)
            wx = xc - x0
            wy = yc - y0
            x0i = jnp.clip(x0, 0, w - 1).astype(jnp.int32)
            x1i = jnp.clip(x0 + 1.0, 0, w - 1).astype(jnp.int32)
            y0i = jnp.clip(y0, 0, h - 1).astype(jnp.int32)
            y1i = jnp.clip(y0 + 1.0, 0, h - 1).astype(jnp.int32)
            base = f * hw
            idx_ref[4 * f + 0] = base + y0i * w + x0i
            idx_ref[4 * f + 1] = base + y0i * w + x1i
            idx_ref[4 * f + 2] = base + y1i * w + x0i
            idx_ref[4 * f + 3] = base + y1i * w + x1i
            wgt_ref[4 * f + 0] = (1 - wx) * (1 - wy)
            wgt_ref[4 * f + 1] = wx * (1 - wy)
            wgt_ref[4 * f + 2] = (1 - wx) * wy
            wgt_ref[4 * f + 3] = wx * wy

    return pl.pallas_call(
        body,
        out_shape=(jax.ShapeDtypeStruct((8, h, w), jnp.int32),
                   jax.ShapeDtypeStruct((8, h, w), _F32)),
        compiler_params=pltpu.CompilerParams(vmem_limit_bytes=_VMEM_LIMIT),
    )(flow_t)


def _sc_gather(table, idx):
    B = idx.shape[0]
    C = table.shape[1]
    info = plsc.get_sparse_core_info()
    nc = info.num_cores
    nw = nc * info.num_subcores
    n = B // nw
    nch = 1
    while True:
        ch = n // nch
        if n % nch == 0 and ch % 128 == 0 and ch * (C + 1) * 4 <= 360_000:
            break
        nch += 1
    chb = ch // 128
    mesh = plsc.VectorSubcoreMesh(core_axis_name="c", subcore_axis_name="s")

    @functools.partial(
        pl.kernel,
        mesh=mesh,
        out_type=jax.ShapeDtypeStruct((B, C), _F32),
        scratch_types=[
            pltpu.VMEM((ch,), jnp.int32),
            pltpu.VMEM((ch, C), _F32),
            pltpu.SemaphoreType.DMA,
        ],
        compiler_params=pltpu.CompilerParams(use_tc_tiling_on_sc=False),
    )
    def k(table_hbm, idx_hbm, out_hbm, idx_v, rows_v, sem):
        wid = lax.axis_index("s") * nc + lax.axis_index("c")
        base = wid * n

        def chunk_body(jc, carry):
            off = base + jc * ch
            pltpu.sync_copy(idx_hbm.at[pl.ds(off, ch)], idx_v)

            def dma_body(j, c2):
                pltpu.async_copy(table_hbm.at[idx_v.at[pl.ds(j * 128, 128)]],
                                 rows_v.at[pl.ds(j * 128, 128)], sem).wait()
                return c2

            lax.fori_loop(0, chb, dma_body, 0)
            pltpu.sync_copy(rows_v, out_hbm.at[pl.ds(off, ch)])
            return carry

        lax.fori_loop(0, nch, chunk_body, 0)

    return k(table, idx)


def _combine(g, wgt, h, w, c):
    th = _row_tile(h)

    def body(g_ref, w_ref, wa_ref, wb_ref):
        wa = g_ref[0] * w_ref[0][:, :, None]
        for k in range(1, 4):
            wa = wa + g_ref[k] * w_ref[k][:, :, None]
        wb = g_ref[4] * w_ref[4][:, :, None]
        for k in range(5, 8):
            wb = wb + g_ref[k] * w_ref[k][:, :, None]
        wa_ref[...] = wa
        wb_ref[...] = wb

    out_sd = jax.ShapeDtypeStruct((h, w, c), _F32)
    return pl.pallas_call(
        body,
        grid=(h // th,),
        in_specs=[pl.BlockSpec((8, th, w, c), lambda i: (0, i, 0, 0)),
                  pl.BlockSpec((8, th, w), lambda i: (0, i, 0))],
        out_specs=(pl.BlockSpec((th, w, c), lambda i: (i, 0, 0)),
                   pl.BlockSpec((th, w, c), lambda i: (i, 0, 0))),
        out_shape=(out_sd, out_sd),
        compiler_params=pltpu.CompilerParams(vmem_limit_bytes=_VMEM_LIMIT),
    )(g, wgt)


def _blend(g, wgt, fo, h, w):
    th = _row_tile(h)

    def body(g_ref, w_ref, f_ref, o_ref):
        w0 = g_ref[0][:, :, :3] * w_ref[0][:, :, None]
        for k in range(1, 4):
            w0 = w0 + g_ref[k][:, :, :3] * w_ref[k][:, :, None]
        w1 = g_ref[4][:, :, :3] * w_ref[4][:, :, None]
        for k in range(5, 8):
            w1 = w1 + g_ref[k][:, :, :3] * w_ref[k][:, :, None]
        m = jax.nn.sigmoid(f_ref[:, :, 0:1])
        res = f_ref[:, :, 1:4]
        o_ref[...] = jnp.clip(w0 * m + w1 * (1.0 - m) + res, 0.0, 1.0)

    return pl.pallas_call(
        body,
        grid=(h // th,),
        in_specs=[pl.BlockSpec((8, th, w, 8), lambda i: (0, i, 0, 0)),
                  pl.BlockSpec((8, th, w), lambda i: (0, i, 0)),
                  pl.BlockSpec((th, w, 4), lambda i: (i, 0, 0))],
        out_specs=pl.BlockSpec((th, w, 3), lambda i: (i, 0, 0)),
        out_shape=jax.ShapeDtypeStruct((h, w, 3), _F32),
        compiler_params=pltpu.CompilerParams(vmem_limit_bytes=_VMEM_LIMIT),
    )(g, wgt, fo)


def _round_up(x, m):
    return (x + m - 1) // m * m


def _warp_pair(fa, fb, flow):
    h, w, c = fa.shape
    hw = h * w
    idx, wgt = _warp_prep(jnp.transpose(flow, (2, 0, 1)), h, w)
    table = jnp.concatenate([fa.reshape(hw, c), fb.reshape(hw, c)], 0)
    b = 8 * hw
    bp = _round_up(b, 4096)
    idx_flat = idx.reshape(b)
    if bp != b:
        idx_flat = jnp.pad(idx_flat, (0, bp - b))
    g = _sc_gather(table, idx_flat)[:b]
    return g.reshape(8, h, w, c), wgt


def _up2(x, method):
    h, w, c = x.shape
    return jax.image.resize(x, (2 * h, 2 * w, c), method=method)


def _glblock(bp, x, mask2d):
    h, w, _ = x.shape
    hcur = _conv3(x, bp['head_w'], None, bp['head_a'], mask2d)
    outs = [hcur]
    for lp in bp['layers']:
        hcur = _conv3(hcur, lp['w'], None, lp['a'], mask2d)
        outs.append(hcur)
    cc = jnp.concatenate(outs, -1)
    cin = cc.shape[-1]
    cout = bp['last_w'].shape[0]
    lw = jnp.transpose(bp['last_w'], (2, 3, 1, 0)).reshape(1, cin, cout)
    hh = _conv_mm(cc, lw, bp['last_b'], None, None, 1, 1, h, w)
    sm = _conv3(hh[..., 4:], bp['mask_w'], bp['mask_b'], None, None)
    h2, w2 = 2 * h, 2 * w
    mask_up = jax.image.resize(mask2d, (h2, w2), method='nearest')
    h_up = _up2(hh, 'bilinear')
    sm_up = _up2(sm, 'bilinear')
    smb = (sm_up[..., 0] > sm_up[..., 1]).astype(_F32) * mask_up
    return h_up[..., :4], h_up[..., 4:], smb


def _encode(enc, img_hwc):
    x = img_hwc
    feats = []
    for i, cp in enumerate(enc):
        if i % 2 == 0:
            x = _conv3_s2(x, cp['w'], cp['b'], cp['a'])
        else:
            x = _conv3(x, cp['w'], cp['b'], cp['a'])
            feats.append(x)
    return feats


def kernel(img0, img1, timestep, params):
    i0 = jnp.transpose(img0[0], (1, 2, 0))
    i1 = jnp.transpose(img1[0], (1, 2, 0))
    f1_0, f2_0, f3_0, f4_0 = _encode(params['enc'], i0)
    f1_1, f2_1, f3_1, f4_1 = _encode(params['enc'], i1)

    h4 = f4_0.shape[0]
    tmap = jnp.broadcast_to(timestep.reshape(1, 1, 1), (h4, h4, 1))
    x = jnp.concatenate([f4_0, f4_1, tmap], -1)
    mask = jnp.ones((h4, h4), _F32)
    flow, feat, mask = _glblock(params['blocks'][0], x, mask)

    for lvl, (fa, fb) in enumerate([(f3_0, f3_1), (f2_0, f2_1), (f1_0, f1_1)]):
        h, w, c = fa.shape
        g, wgt = _warp_pair(fa, fb, flow)
        wa, wb = _combine(g, wgt, h, w, c)
        x = jnp.concatenate([wa, wb, feat, flow], -1)
        flow, feat, mask = _glblock(params['blocks'][lvl + 1], x, mask)

    fo = _conv3(feat, params['final']['w'], params['final']['b'], None, None)
    hh = fo.shape[0]
    i0p = jnp.pad(i0, ((0, 0), (0, 0), (0, 5)))
    i1p = jnp.pad(i1, ((0, 0), (0, 0), (0, 5)))
    g, wgt = _warp_pair(i0p, i1p, flow)
    pred = _blend(g, wgt, fo, hh, hh)
    return jnp.transpose(pred, (2, 0, 1))[None]
